# baseline (speedup 1.0000x reference)
import functools

import jax
import jax.numpy as jnp
from jax import lax
from jax.experimental import pallas as pl
from jax.experimental.pallas import tpu as pltpu
from jax.experimental.pallas import tpu_sc as plsc

_E = 160000
_T = 320000
_C = 128

_NC = 2
_NS = 16
_R = 14080
_ACC_ROWS = _R + 128
_HALF = _E // _NC
_NPASS = -(-_HALF // _R)
_TILE_T = _T // _NS
_CH = 1024
_GB = 128
_ZROWS = _ACC_ROWS // _NS

_BR = 1600


def _dot(a, b):
    return jax.lax.dot(a.astype(jnp.bfloat16), b.astype(jnp.bfloat16),
                       preferred_element_type=jnp.float32)


def _mlp_body(x_ref, w1_ref, b1_ref, w2_ref, b2_ref, o_ref):
    x = x_ref[...]
    h = jnp.maximum(_dot(x, w1_ref[...]) + b1_ref[...], 0.0)
    o_ref[...] = _dot(h, w2_ref[...]) + b2_ref[...]


def _mlp(x, w1, b1, w2, b2):
    e, c = x.shape
    h = w1.shape[1]
    return pl.pallas_call(
        _mlp_body,
        grid=(e // _BR,),
        in_specs=[
            pl.BlockSpec((_BR, c), lambda i: (i, 0)),
            pl.BlockSpec((c, h), lambda i: (0, 0)),
            pl.BlockSpec((h,), lambda i: (0,)),
            pl.BlockSpec((h, c), lambda i: (0, 0)),
            pl.BlockSpec((c,), lambda i: (0,)),
        ],
        out_specs=pl.BlockSpec((_BR, c), lambda i: (i, 0)),
        out_shape=jax.ShapeDtypeStruct((e, c), jnp.float32),
    )(x, w1, b1, w2, b2)


def _linear_body(x_ref, w_ref, b_ref, o_ref):
    x = jnp.maximum(x_ref[...], 0.0)
    o_ref[...] = _dot(x, w_ref[...]) + b_ref[...]


def _relu_linear(x, w, b):
    e, c = x.shape
    return pl.pallas_call(
        _linear_body,
        grid=(e // _BR,),
        in_specs=[
            pl.BlockSpec((_BR, c), lambda i: (i, 0)),
            pl.BlockSpec((c, c), lambda i: (0, 0)),
            pl.BlockSpec((c,), lambda i: (0,)),
        ],
        out_specs=pl.BlockSpec((_BR, c), lambda i: (i, 0)),
        out_shape=jax.ShapeDtypeStruct((e, c), jnp.float32),
    )(x, w, b)


_PB = _T // _GB
_NW = _NC * _NS


def _prod_body(ha_hbm, ia_hbm, hb_hbm, ib_hbm, p_hbm,
               iabuf, ibbuf, rows_a, rows_b, sem_a, sem_b):
    cid = lax.axis_index("c")
    sid = lax.axis_index("s")
    w = sid * _NC + cid
    nb = _PB // _NW + (w < (_PB - (_PB // _NW) * _NW)).astype(jnp.int32)

    def gbody(k, _):
        off = (w + k * _NW) * _GB
        pltpu.sync_copy(ia_hbm.at[pl.ds(off, _GB)], iabuf)
        pltpu.sync_copy(ib_hbm.at[pl.ds(off, _GB)], ibbuf)
        cpa = pltpu.async_copy(ha_hbm.at[iabuf], rows_a, sem_a)
        cpb = pltpu.async_copy(hb_hbm.at[ibbuf], rows_b, sem_b)
        cpa.wait()
        cpb.wait()

        def mbody(r, _):
            for j in range(_C // 16):
                s = pl.ds(j * 16, 16)
                rows_a[r, s] = rows_a[r, s] * rows_b[r, s]
            return 0

        lax.fori_loop(0, _GB, mbody, 0)
        pltpu.sync_copy(rows_a, p_hbm.at[pl.ds(off, _GB)])
        return 0

    lax.fori_loop(0, nb, gbody, 0)


def _scat_body(p_hbm, id_hbm, out_hbm,
               acc, pbuf, ddraw, dd1, sem_s):
    cid = lax.axis_index("c")
    sid = lax.axis_index("s")
    core_base = cid * _HALF
    nb = _PB // _NS + (sid < (_PB - (_PB // _NS) * _NS)).astype(jnp.int32)

    def pass_body(p, _):
        base = core_base + p * _R

        def zrow(r, _):
            for j in range(_C // 16):
                pbuf[r, pl.ds(j * 16, 16)] = jnp.zeros((16,), jnp.float32)
            return 0

        lax.fori_loop(0, _GB, zrow, 0)

        def zcp(z, _):
            pltpu.sync_copy(pbuf, acc.at[pl.ds(sid * _ZROWS + z * _GB, _GB)])
            return 0

        lax.fori_loop(0, _ZROWS // _GB, zcp, 0)
        pltpu.sync_copy(pbuf.at[pl.ds(0, _ZROWS - (_ZROWS // _GB) * _GB)],
                        acc.at[pl.ds(sid * _ZROWS + (_ZROWS // _GB) * _GB,
                                     _ZROWS - (_ZROWS // _GB) * _GB)])
        plsc.subcore_barrier()

        def gbody(k, _):
            off = (sid + k * _NS) * _GB
            pltpu.sync_copy(id_hbm.at[pl.ds(off, _GB)], ddraw)
            pltpu.sync_copy(p_hbm.at[pl.ds(off, _GB)], pbuf)
            for j in range(_GB // 16):
                s = pl.ds(j * 16, 16)
                rel = ddraw[s] - core_base
                loc = rel - p * _R
                m = (loc >= 0) & (loc < _R) & (rel < _HALF)
                dd1[s] = jnp.where(m, loc, _R)
            pltpu.sync_copy(pbuf, acc.at[dd1], add=True)
            return 0

        lax.fori_loop(0, nb, gbody, 0)
        plsc.subcore_barrier()

        def fchunk(r0, rows):
            pltpu.sync_copy(acc.at[pl.ds(r0, rows)], pbuf.at[pl.ds(0, rows)])
            pltpu.sync_copy(pbuf.at[pl.ds(0, rows)],
                            out_hbm.at[pl.ds(base + r0, rows)])

        vr16 = jnp.where(p == _NPASS - 1, (_HALF - (_NPASS - 1) * _R) // _NS,
                         _R // _NS)
        r0base = sid * vr16

        def fbody(f, _):
            fchunk(r0base + f * 128, 128)
            return 0

        lax.fori_loop(0, vr16 // 128, fbody, 0)

        @pl.when(p < _NPASS - 1)
        def _():
            fchunk(r0base + 6 * 128, 112)

        @pl.when(p == _NPASS - 1)
        def _():
            fchunk(r0base + 4 * 128, 88)

        plsc.subcore_barrier()
        return 0

    lax.fori_loop(0, _NPASS, pass_body, 0)


def _segsum(ha, ia, hb, ib, idst):
    mesh = plsc.VectorSubcoreMesh(core_axis_name="c", subcore_axis_name="s")
    prod = pl.kernel(
        _prod_body,
        out_type=jax.ShapeDtypeStruct((_T, _C), jnp.float32),
        mesh=mesh,
        scratch_types=[
            pltpu.VMEM((_GB,), jnp.int32),
            pltpu.VMEM((_GB,), jnp.int32),
            pltpu.VMEM((_GB, _C), jnp.float32),
            pltpu.VMEM((_GB, _C), jnp.float32),
            pltpu.SemaphoreType.DMA,
            pltpu.SemaphoreType.DMA,
        ],
    )(ha, ia, hb, ib)
    mesh2 = plsc.VectorSubcoreMesh(core_axis_name="c", subcore_axis_name="s")
    return pl.kernel(
        _scat_body,
        out_type=jax.ShapeDtypeStruct((_E, _C), jnp.float32),
        mesh=mesh2,
        scratch_types=[
            pltpu.VMEM_SHARED((_ACC_ROWS, _C), jnp.float32),
            pltpu.VMEM((_GB, _C), jnp.float32),
            pltpu.VMEM((_GB,), jnp.int32),
            pltpu.VMEM((_GB,), jnp.int32),
            pltpu.SemaphoreType.DMA,
        ],
    )(prod, idst)


_CCH = 128
_NCHUNK = _E // _CCH


def _combine_body(e_hbm, msa_hbm, msb_hbm, msc_hbm, inv_hbm, out_hbm,
                  accb, tmpb, gbuf, ibuf, sem_g):
    cid = lax.axis_index("c")
    sid = lax.axis_index("s")
    widx = sid * _NC + cid

    def chunk(chi, _):
        off = chi * _CCH
        sl = pl.ds(off, _CCH)
        pltpu.sync_copy(e_hbm.at[sl], accb)
        pltpu.sync_copy(inv_hbm.at[sl], ibuf)
        cpg = pltpu.async_copy(msb_hbm.at[ibuf], gbuf, sem_g)

        def add_from(src_hbm):
            pltpu.sync_copy(src_hbm.at[sl], tmpb)

            def abody(r, _):
                for j in range(_C // 16):
                    s = pl.ds(j * 16, 16)
                    accb[r, s] = accb[r, s] + tmpb[r, s]
                return 0

            lax.fori_loop(0, _CCH, abody, 0)

        add_from(msa_hbm)
        add_from(msb_hbm)
        add_from(msc_hbm)
        cpg.wait()

        def gaddbody(r, _):
            for j in range(_C // 16):
                s = pl.ds(j * 16, 16)
                accb[r, s] = accb[r, s] + gbuf[r, s]
            return 0

        lax.fori_loop(0, _CCH, gaddbody, 0)
        pltpu.sync_copy(accb, out_hbm.at[sl])
        return 0

    nfull = _NCHUNK // (_NC * _NS)
    nrem = _NCHUNK - nfull * _NC * _NS

    def round_body(k, _):
        return chunk(widx + k * _NC * _NS, 0)

    lax.fori_loop(0, nfull, round_body, 0)

    @pl.when(widx < nrem)
    def _():
        chunk(nfull * _NC * _NS + widx, 0)


def _combine(e, msa, msb, msc, inv):
    mesh = plsc.VectorSubcoreMesh(core_axis_name="c", subcore_axis_name="s")
    return pl.kernel(
        _combine_body,
        out_type=jax.ShapeDtypeStruct((_E, _C), jnp.float32),
        mesh=mesh,
        scratch_types=[
            pltpu.VMEM((_CCH, _C), jnp.float32),
            pltpu.VMEM((_CCH, _C), jnp.float32),
            pltpu.VMEM((_CCH, _C), jnp.float32),
            pltpu.VMEM((_CCH,), jnp.int32),
            pltpu.SemaphoreType.DMA,
        ],
    )(e, msa, msb, msc, inv)


def kernel(edge_attr, edge_attr2, triangle_1_1_1, triangle_1_1_2, triangle_1_2_2,
           triangle_2_2_2, inverse_edge_1, inverse_edge_2,
           mlp_W1, mlp_b1, mlp_W2, mlp_b2, lin_W1, lin_b1, lin_W2, lin_b2):
    ij111, ik111, kj111 = triangle_1_1_1[0], triangle_1_1_1[1], triangle_1_1_1[2]
    ij112, ik112, kj112 = triangle_1_1_2[0], triangle_1_1_2[1], triangle_1_1_2[2]
    ij122, ik122, kj122 = triangle_1_2_2[0], triangle_1_2_2[1], triangle_1_2_2[2]
    ij222, ik222, kj222 = triangle_2_2_2[0], triangle_2_2_2[1], triangle_2_2_2[2]

    def m(i, x):
        return _mlp(x, mlp_W1[0, i], mlp_b1[0, i], mlp_W2[0, i], mlp_b2[0, i])

    h0 = m(0, edge_attr)
    h1 = m(1, edge_attr)
    h2 = m(2, edge_attr2)
    h3 = m(3, edge_attr2)
    h6 = m(6, edge_attr2)
    h7 = m(7, edge_attr2)

    ms111 = _segsum(h0, ik111, h0, kj111, ij111)
    ms112 = _segsum(h1, ik112, h2, kj112, ij112)
    ms122 = _segsum(h3, ik122, h3, kj122, ij122)
    eu = _combine(edge_attr, ms111, ms112, ms122, inverse_edge_1)

    h4 = m(4, eu)
    h5 = m(5, eu)

    ms211 = _segsum(h4, ij112, h4, ik112, kj112)
    ms212 = _segsum(h5, ij122, h6, kj122, ik122)
    ms222 = _segsum(h7, ik222, h7, kj222, ij222)
    eu2 = _combine(edge_attr2, ms211, ms212, ms222, inverse_edge_2)

    out1 = _relu_linear(eu, lin_W1, lin_b1)
    out2 = _relu_linear(eu2, lin_W2, lin_b2)
    return (out1, out2)
